# baseline (speedup 1.0000x reference)
import functools

import jax
import jax.numpy as jnp
from jax import lax
from jax.experimental import pallas as pl
from jax.experimental.pallas import tpu as pltpu
from jax.experimental.pallas import tpu_sc as plsc

_N_FIELDS = 26
_VOCAB = 100000
_EMB = 32
_B = 16384
_H1 = 512
_H2 = 256
_OUT = 1

_NIDX = _B * _N_FIELDS
_ROWS4 = _N_FIELDS * _VOCAB // 4
_W = 128
_WPS = 8
_NW = 32
_SPT = _NIDX // (_NW * _WPS * _W)


def _sc_gather(tab128, tid2d, sel2d):
    mesh = plsc.VectorSubcoreMesh(core_axis_name="c", subcore_axis_name="s")

    @functools.partial(
        pl.kernel,
        out_type=jax.ShapeDtypeStruct((_NIDX, _EMB), jnp.float32),
        mesh=mesh,
        scratch_types=[
            pltpu.VMEM((_WPS, _W), jnp.int32),
            pltpu.VMEM((_WPS, _W), jnp.int32),
            pltpu.VMEM((_W, 128), jnp.float32),
            pltpu.VMEM((_W, 128), jnp.float32),
            pltpu.VMEM((_WPS * _W, _EMB), jnp.float32),
            pltpu.SemaphoreType.DMA,
            pltpu.SemaphoreType.DMA,
        ],
        compiler_params=pltpu.CompilerParams(
            use_tc_tiling_on_sc=False, needs_layout_passes=False
        ),
    )
    def gather_kernel(tab_hbm, tid_hbm, sel_hbm, out_hbm,
                      tid_v, sel_v, fetch_a, fetch_b, o_buf, osem, gsem):
        wid = lax.axis_index("s") * 2 + lax.axis_index("c")
        row0 = wid * (_SPT * _WPS)
        fetches = (fetch_a, fetch_b)

        @pl.loop(0, _SPT)
        def _superstep(it):
            base_row = row0 + it * _WPS
            pltpu.sync_copy(tid_hbm.at[pl.ds(base_row, _WPS)], tid_v)
            pltpu.sync_copy(sel_hbm.at[pl.ds(base_row, _WPS)], sel_v)
            c0 = pltpu.async_copy(tab_hbm.at[tid_v.at[0]], fetches[0], gsem)
            pending = c0
            for w in range(_WPS):
                fetch_v = fetches[w % 2]
                pending.wait()
                if w + 1 < _WPS:
                    pending = pltpu.async_copy(
                        tab_hbm.at[tid_v.at[w + 1]], fetches[(w + 1) % 2], gsem
                    )
                for k in range(0, _W, 16):
                    i0 = lax.iota(jnp.int32, 16) + k
                    s32 = sel_v[w, pl.ds(k, 16)] * _EMB
                    o0 = i0 + (w * _W)
                    for cc in range(_EMB):
                        c16 = jnp.full((16,), cc, jnp.int32)
                        vals = plsc.load_gather(fetch_v, [i0, s32 + c16])
                        plsc.store_scatter(o_buf, [o0, c16], vals)
            out_base = row0 * _W + it * (_WPS * _W)
            pltpu.sync_copy(o_buf, out_hbm.at[pl.ds(out_base, _WPS * _W)])

    return gather_kernel(tab128, tid2d, sel2d)


_BM = 1024


def _mlp_body(x_ref, w1_ref, b1_ref, w2_ref, b2_ref, w3_ref, b3_ref, o_ref):
    cdims = (((1,), (1,)), ((), ()))
    x = x_ref[...]
    h = lax.dot_general(x, w1_ref[...], cdims, preferred_element_type=jnp.float32)
    h = jnp.maximum(h + b1_ref[...], 0.0)
    h = lax.dot_general(h, w2_ref[...], cdims, preferred_element_type=jnp.float32)
    h = jnp.maximum(h + b2_ref[...], 0.0)
    o = jnp.sum(h * w3_ref[...], axis=1, keepdims=True)
    o = o + b3_ref[0, 0]
    o_ref[...] = jax.nn.sigmoid(o)


def _tc_mlp(x, W1, b1, W2, b2, W3, b3):
    n_embs = _N_FIELDS * _EMB
    grid = (_B // _BM,)
    return pl.pallas_call(
        _mlp_body,
        grid=grid,
        in_specs=[
            pl.BlockSpec((_BM, n_embs), lambda i: (i, 0)),
            pl.BlockSpec((_H1, n_embs), lambda i: (0, 0)),
            pl.BlockSpec((1, _H1), lambda i: (0, 0)),
            pl.BlockSpec((_H2, _H1), lambda i: (0, 0)),
            pl.BlockSpec((1, _H2), lambda i: (0, 0)),
            pl.BlockSpec((_OUT, _H2), lambda i: (0, 0)),
            pl.BlockSpec((1, _OUT), lambda i: (0, 0)),
        ],
        out_specs=pl.BlockSpec((_BM, _OUT), lambda i: (i, 0)),
        out_shape=jax.ShapeDtypeStruct((_B, _OUT), jnp.float32),
    )(x, W1, b1.reshape(1, _H1), W2, b2.reshape(1, _H2), W3, b3.reshape(1, _OUT))


def kernel(cat_data, tables, W1, b1, W2, b2, W3, b3):
    tab128 = tables.reshape(_ROWS4, 128)
    offs = (jnp.arange(_N_FIELDS, dtype=jnp.int32) * _VOCAB)[None, :]
    idx = (cat_data + offs).reshape(_NIDX // _W, _W)
    tid2d = jax.lax.shift_right_logical(idx, 2)
    sel2d = jax.lax.bitwise_and(idx, 3)
    gathered = _sc_gather(tab128, tid2d, sel2d)
    x = gathered.reshape(_B, _N_FIELDS * _EMB)
    return _tc_mlp(x, W1, b1, W2, b2, W3, b3)
